# baseline (speedup 1.0000x reference)
import functools

import jax
import jax.numpy as jnp
from jax import lax
from jax.experimental import pallas as pl
from jax.experimental.pallas import tpu as pltpu
from jax.experimental.pallas import tpu_sc as plsc


def _iota16():
    return lax.iota(jnp.int32, 16)


def _splat(x):
    return jnp.full((16,), x, dtype=jnp.int32)


@functools.lru_cache(maxsize=None)
def _build(b_sz: int, l_sz: int, vocab: int, hidden: int):
    info = plsc.get_sparse_core_info()
    nc, ns = info.num_cores, info.num_subcores
    nw = nc * ns
    assert hidden == 64 and b_sz % (nw * 128) == 0

    npair = vocab // 2
    nfull = vocab // 128
    tail_w = vocab - nfull * 128
    base_k, extra = divmod(nfull, nw)

    mesh = plsc.VectorSubcoreMesh(core_axis_name="c", subcore_axis_name="s")

    n_tail_pair = tail_w // 2

    @functools.partial(
        pl.kernel,
        mesh=mesh,
        compiler_params=pltpu.CompilerParams(needs_layout_passes=False),
        out_type=jax.ShapeDtypeStruct((npair, 128), jnp.float32),
        scratch_types=[
            pltpu.VMEM((hidden, 128), jnp.float32),
            pltpu.VMEM((hidden, 128), jnp.float32),
        ],
    )
    def pack_kernel(tt_hbm, tail_hbm, pairs_hbm, chunk_v, pack_v):
        wid = lax.axis_index("s") * nc + lax.axis_index("c")
        hvec = [_iota16() + (k0 * 16) for k0 in range(4)]

        def do_chunk(c, carry):
            pltpu.sync_copy(tt_hbm.at[:, pl.ds(c * 128, 128)], chunk_v)

            def row(j, carry2):
                v0 = _splat(2 * j)
                v1 = _splat(2 * j + 1)
                for k in range(4):
                    val = plsc.load_gather(chunk_v, [hvec[k], v0])
                    pack_v[j, pl.ds(k * 16, 16)] = val
                for k in range(4):
                    val = plsc.load_gather(chunk_v, [hvec[k], v1])
                    pack_v[j, pl.ds(64 + k * 16, 16)] = val
                return carry2

            lax.fori_loop(0, 64, row, 0)
            pltpu.sync_copy(pack_v, pairs_hbm.at[pl.ds(c * 64, 64), :])
            return carry

        n_k = jnp.where(wid < extra, base_k + 1, base_k)

        def chunk_iter(k, carry):
            return do_chunk(k * nw + wid, carry)

        lax.fori_loop(0, n_k, chunk_iter, 0)

        if n_tail_pair:
            @pl.when(wid == nw - 1)
            def _():
                pltpu.sync_copy(tail_hbm, pack_v.at[pl.ds(0, n_tail_pair), :])
                pltpu.sync_copy(pack_v.at[pl.ds(0, n_tail_pair), :],
                                pairs_hbm.at[pl.ds(npair - n_tail_pair,
                                                   n_tail_pair), :])

    bpw = b_sz // nw

    @functools.partial(
        pl.kernel,
        mesh=mesh,
        compiler_params=pltpu.CompilerParams(needs_layout_passes=False),
        out_type=jax.ShapeDtypeStruct((l_sz, hidden, b_sz), jnp.float32),
        scratch_types=[
            pltpu.VMEM((l_sz, bpw), jnp.int32),
            pltpu.VMEM((8, bpw), jnp.int32),
            pltpu.VMEM((bpw, 128), jnp.float32),
            pltpu.VMEM((hidden, bpw), jnp.float32),
            pltpu.SemaphoreType.DMA,
        ],
    )
    def gather_kernel(pairs_hbm, idxt_hbm, out_hbm, idx_v, plist_v,
                      grows_v, oblk_v, gsem):
        wid = lax.axis_index("s") * nc + lax.axis_index("c")
        b0 = wid * bpw
        pltpu.sync_copy(idxt_hbm.at[:, pl.ds(b0, bpw)], idx_v)
        rowvec = [_iota16() + (kb * 16) for kb in range(8)]

        def block(l, carry):
            pcol = []
            for kb in range(8):
                ids = idx_v[l, pl.ds(kb * 16, 16)]
                plist_v[0, pl.ds(kb * 16, 16)] = lax.shift_right_logical(ids, 1)
                pcol.append(lax.shift_left((ids & 1), 6))
            pltpu.async_copy(pairs_hbm.at[plist_v.at[0]], grows_v, gsem).wait()

            def hrow(h, carry2):
                for kb in range(8):
                    col = pcol[kb] + h
                    val = plsc.load_gather(grows_v, [rowvec[kb], col])
                    oblk_v[h, pl.ds(kb * 16, 16)] = val
                return carry2

            lax.fori_loop(0, hidden, hrow, 0)
            pltpu.sync_copy(oblk_v, out_hbm.at[l, :, pl.ds(b0, bpw)])
            return carry

        lax.fori_loop(0, l_sz, block, 0)

    return pack_kernel, gather_kernel


def kernel(input_ids, table):
    b_sz, l_sz = input_ids.shape
    vocab, hidden = table.shape
    tt = jnp.transpose(table)
    idx_t = jnp.transpose(input_ids.astype(jnp.int32))
    pack_kernel, gather_kernel = _build(b_sz, l_sz, vocab, hidden)
    nfull = vocab // 128
    tail = jnp.reshape(table[nfull * 128:], ((vocab - nfull * 128) // 2, 128))
    pairs = pack_kernel(tt, tail)
    out_t = gather_kernel(pairs, idx_t)
    return jnp.transpose(out_t, (2, 0, 1))


# speedup vs baseline: 2.7398x; 2.7398x over previous
import functools

import jax
import jax.numpy as jnp
from jax import lax
from jax.experimental import pallas as pl
from jax.experimental.pallas import tpu as pltpu
from jax.experimental.pallas import tpu_sc as plsc


def _iota16():
    return lax.iota(jnp.int32, 16)


@functools.lru_cache(maxsize=None)
def _build(b_sz: int, l_sz: int, vocab: int, hidden: int):
    info = plsc.get_sparse_core_info()
    nc, ns = info.num_cores, info.num_subcores
    nw = nc * ns
    assert hidden == 64 and vocab % 2 == 0 and b_sz % (nw * 128) == 0
    npair = vocab // 2
    bpw = b_sz // nw
    n_blk = l_sz

    mesh = plsc.VectorSubcoreMesh(core_axis_name="c", subcore_axis_name="s")

    @functools.partial(
        pl.kernel,
        mesh=mesh,
        compiler_params=pltpu.CompilerParams(needs_layout_passes=False),
        out_type=jax.ShapeDtypeStruct((l_sz, hidden, b_sz), jnp.float32),
        scratch_types=[
            pltpu.VMEM((l_sz, bpw), jnp.int32),
            pltpu.VMEM((2, bpw), jnp.int32),
            pltpu.VMEM((bpw, 128), jnp.float32),
            pltpu.VMEM((bpw, 128), jnp.float32),
            pltpu.VMEM((hidden, bpw), jnp.float32),
            pltpu.VMEM((hidden, bpw), jnp.float32),
            pltpu.SemaphoreType.DMA,
            pltpu.SemaphoreType.DMA,
            pltpu.SemaphoreType.DMA,
            pltpu.SemaphoreType.DMA,
        ],
    )
    def gather_kernel(pairs_hbm, idxt_hbm, out_hbm, idx_v, plist_v,
                      grows0, grows1, oblk0, oblk1,
                      gsem0, gsem1, ssem0, ssem1):
        wid = lax.axis_index("s") * nc + lax.axis_index("c")
        b0 = wid * bpw
        grows = (grows0, grows1)
        oblk = (oblk0, oblk1)
        gsem = (gsem0, gsem1)
        ssem = (ssem0, ssem1)
        rowvec = [_iota16() + (kb * 16) for kb in range(8)]

        pltpu.sync_copy(idxt_hbm.at[:, pl.ds(b0, bpw)], idx_v)

        def fire_gather(i, b):
            for kb in range(8):
                ids = idx_v[i, pl.ds(kb * 16, 16)]
                plist_v[b, pl.ds(kb * 16, 16)] = lax.shift_right_logical(ids, 1)
            pltpu.async_copy(pairs_hbm.at[plist_v.at[b]], grows[b], gsem[b])

        def wait_gather(b):
            pltpu.make_async_copy(pairs_hbm.at[pl.ds(0, bpw)],
                                  grows[b], gsem[b]).wait()

        def extract(i, b):
            pcol = []
            for kb in range(8):
                ids = idx_v[i, pl.ds(kb * 16, 16)]
                pcol.append(lax.shift_left(ids & 1, 6))
            src = grows[b]
            dst = oblk[b]

            @plsc.parallel_loop(0, hidden, unroll=4)
            def hrow(h):
                for kb in range(8):
                    val = plsc.load_gather(src, [rowvec[kb], pcol[kb] + h])
                    dst[h, pl.ds(kb * 16, 16)] = val

        def fire_store(i, b):
            pltpu.async_copy(oblk[b], out_hbm.at[i, :, pl.ds(b0, bpw)],
                             ssem[b])

        def wait_store(b):
            pltpu.make_async_copy(oblk[b], out_hbm.at[0, :, pl.ds(b0, bpw)],
                                  ssem[b]).wait()

        fire_gather(0, 0)

        def do_block(i, b, first, last):
            if not first:
                wait_store(b)
            if not last:
                fire_gather(i + 1, 1 - b)
            wait_gather(b)
            extract(i, b)
            fire_store(i, b)

        do_block(0, 0, True, False)
        do_block(1, 1, True, False)

        def step(s, carry):
            i0 = s * 2
            do_block(i0, 0, False, False)
            do_block(i0 + 1, 1, False, False)
            return carry

        lax.fori_loop(1, n_blk // 2 - 1, step, 0)

        do_block(n_blk - 2, 0, False, False)
        do_block(n_blk - 1, 1, False, True)
        wait_store(0)
        wait_store(1)

    return gather_kernel


def kernel(input_ids, table):
    b_sz, l_sz = input_ids.shape
    vocab, hidden = table.shape
    tpair = jnp.reshape(table, (vocab // 2, 2 * hidden))
    idx_t = jnp.transpose(input_ids.astype(jnp.int32))
    fn = _build(b_sz, l_sz, vocab, hidden)
    out_t = fn(tpair, idx_t)
    return jnp.transpose(out_t, (2, 0, 1))
